# baseline (speedup 1.0000x reference)
import functools
import math

import jax
import jax.numpy as jnp
from jax import lax
from jax.experimental import pallas as pl
from jax.experimental.pallas import tpu as pltpu
from jax.experimental.pallas import tpu_sc as plsc

D_MODEL = 64
LANES = 16
NUM_CORES = 2
NUM_SUBCORES = 16
NUM_WORKERS = NUM_CORES * NUM_SUBCORES
SCALE = math.sqrt(D_MODEL)

BBLK = 128
NBUF = 2


def _emb_body(seq_len, idx_hbm, table_hbm, out_hbm,
              idx_v, in0, in1, st0, st1, gs0, gs1, ss0, ss1):
    wid = lax.axis_index("s") * NUM_CORES + lax.axis_index("c")

    ins = (in0, in1)
    stages = (st0, st1)
    gsems = (gs0, gs1)
    ssems = (ss0, ss1)

    pltpu.sync_copy(idx_hbm.at[wid], idx_v)

    def gather_desc(s, b):
        return pltpu.make_async_copy(
            table_hbm.at[idx_v.at[s]], ins[b], gsems[b])

    def store_desc(s, b):
        return pltpu.make_async_copy(
            stages[b], out_hbm.at[s, :, wid], ssems[b])

    for b in range(NBUF):
        gather_desc(b, b).start()

    def outer(s0, _):
        for b in range(NBUF):
            s = s0 * NBUF + b
            inb, stb = ins[b], stages[b]
            gather_desc(s, b).wait()

            @pl.when(s >= NBUF)
            def _():
                store_desc(s - NBUF, b).wait()

            def col_body(cg, _):
                col0 = cg * 8
                for c8 in range(8):
                    col = jnp.full((LANES,), col0 + c8, jnp.int32)
                    for lg in range(BBLK // LANES):
                        row = lax.iota(jnp.int32, LANES) + (lg * LANES)
                        v = plsc.load_gather(inb, [row, col])
                        stb[cg, c8, pl.ds(lg * LANES, LANES)] = v * SCALE
                return ()

            lax.fori_loop(0, D_MODEL // 8, col_body, ())

            @pl.when(s + NBUF < seq_len)
            def _():
                gather_desc(s + NBUF, b).start()
            store_desc(s, b).start()
        return ()

    lax.fori_loop(0, seq_len // NBUF, outer, ())

    for b in range(NBUF):
        store_desc(seq_len - NBUF + b, b).wait()


@functools.partial(jax.jit, static_argnames=("seq_len",))
def _emb_call(idx, table, seq_len):
    mesh = plsc.VectorSubcoreMesh(core_axis_name="c", subcore_axis_name="s")
    return pl.kernel(
        functools.partial(_emb_body, seq_len),
        mesh=mesh,
        out_type=jax.ShapeDtypeStruct(
            (seq_len, D_MODEL // 8, NUM_WORKERS, 8, BBLK), jnp.float32),
        scratch_types=[
            pltpu.VMEM((seq_len, BBLK), jnp.int32),
            pltpu.VMEM((BBLK, D_MODEL), jnp.float32),
            pltpu.VMEM((BBLK, D_MODEL), jnp.float32),
            pltpu.VMEM((D_MODEL // 8, 8, BBLK), jnp.float32),
            pltpu.VMEM((D_MODEL // 8, 8, BBLK), jnp.float32),
            pltpu.SemaphoreType.DMA,
            pltpu.SemaphoreType.DMA,
            pltpu.SemaphoreType.DMA,
            pltpu.SemaphoreType.DMA,
        ],
        compiler_params=pltpu.CompilerParams(use_tc_tiling_on_sc=False,
                                             needs_layout_passes=False),
    )(idx, table)


def kernel(x, table):
    bsz, seq_len = x.shape
    idx = x.reshape(NUM_WORKERS, BBLK, seq_len).transpose(0, 2, 1)
    out5 = _emb_call(idx, table, seq_len)
    return out5.transpose(2, 4, 0, 1, 3).reshape(bsz, seq_len, D_MODEL)


# speedup vs baseline: 1.8009x; 1.8009x over previous
import functools
import math

import jax
import jax.numpy as jnp
from jax import lax
from jax.experimental import pallas as pl
from jax.experimental.pallas import tpu as pltpu
from jax.experimental.pallas import tpu_sc as plsc

D_MODEL = 64
LANES = 16
NUM_CORES = 2
NUM_SUBCORES = 16
NUM_WORKERS = NUM_CORES * NUM_SUBCORES
SCALE = math.sqrt(D_MODEL)

BBLK = 128
PITCH = BBLK + 1
NBUF = 2


def _emb_body(seq_len, idx_hbm, table_hbm, out_hbm,
              idx_v, in0, in1, st0, st1, gs0, gs1, ss0, ss1):
    wid = lax.axis_index("s") * NUM_CORES + lax.axis_index("c")

    ins = (in0, in1)
    stages = (st0, st1)
    gsems = (gs0, gs1)
    ssems = (ss0, ss1)

    pltpu.sync_copy(idx_hbm.at[wid], idx_v)

    def gather_desc(s, b):
        return pltpu.make_async_copy(
            table_hbm.at[idx_v.at[s]], ins[b], gsems[b])

    def store_desc(s, b):
        return pltpu.make_async_copy(
            stages[b].at[:, :, pl.ds(0, BBLK)], out_hbm.at[s, :, wid],
            ssems[b])

    for b in range(NBUF):
        gather_desc(b, b).start()

    iota = lax.iota(jnp.int32, LANES)
    cgs = [(iota + 16 * j) >> 3 for j in range(D_MODEL // LANES)]
    c8s = [(iota + 16 * j) & 7 for j in range(D_MODEL // LANES)]

    def outer(s0, _):
        for b in range(NBUF):
            s = s0 * NBUF + b
            inb, stb = ins[b], stages[b]
            gather_desc(s, b).wait()

            @pl.when(s >= NBUF)
            def _():
                store_desc(s - NBUF, b).wait()

            def row_body(l, _):
                lane_l = jnp.full((LANES,), l, jnp.int32)
                for j in range(D_MODEL // LANES):
                    v = inb[l, pl.ds(j * LANES, LANES)] * SCALE
                    plsc.store_scatter(stb, [cgs[j], c8s[j], lane_l], v)
                return ()

            lax.fori_loop(0, BBLK, row_body, (), unroll=4)

            @pl.when(s + NBUF < seq_len)
            def _():
                gather_desc(s + NBUF, b).start()
            store_desc(s, b).start()
        return ()

    lax.fori_loop(0, seq_len // NBUF, outer, ())

    for b in range(NBUF):
        store_desc(seq_len - NBUF + b, b).wait()


@functools.partial(jax.jit, static_argnames=("seq_len",))
def _emb_call(idx, table, seq_len):
    mesh = plsc.VectorSubcoreMesh(core_axis_name="c", subcore_axis_name="s")
    return pl.kernel(
        functools.partial(_emb_body, seq_len),
        mesh=mesh,
        out_type=jax.ShapeDtypeStruct(
            (seq_len, D_MODEL // 8, NUM_WORKERS, 8, BBLK), jnp.float32),
        scratch_types=[
            pltpu.VMEM((seq_len, BBLK), jnp.int32),
            pltpu.VMEM((BBLK, D_MODEL), jnp.float32),
            pltpu.VMEM((BBLK, D_MODEL), jnp.float32),
            pltpu.VMEM((D_MODEL // 8, 8, PITCH), jnp.float32),
            pltpu.VMEM((D_MODEL // 8, 8, PITCH), jnp.float32),
            pltpu.SemaphoreType.DMA,
            pltpu.SemaphoreType.DMA,
            pltpu.SemaphoreType.DMA,
            pltpu.SemaphoreType.DMA,
        ],
        compiler_params=pltpu.CompilerParams(use_tc_tiling_on_sc=False,
                                             needs_layout_passes=False),
    )(idx, table)


def kernel(x, table):
    bsz, seq_len = x.shape
    idx = x.reshape(NUM_WORKERS, BBLK, seq_len).transpose(0, 2, 1)
    out5 = _emb_call(idx, table, seq_len)
    return out5.transpose(2, 4, 0, 1, 3).reshape(bsz, seq_len, D_MODEL)


# speedup vs baseline: 2.5601x; 1.4216x over previous
import functools
import math

import jax
import jax.numpy as jnp
from jax import lax
from jax.experimental import pallas as pl
from jax.experimental.pallas import tpu as pltpu
from jax.experimental.pallas import tpu_sc as plsc

D_MODEL = 64
LANES = 16
NUM_CORES = 2
NUM_SUBCORES = 16
NUM_WORKERS = NUM_CORES * NUM_SUBCORES
SCALE = math.sqrt(D_MODEL)

BBLK = 128
PITCH = BBLK + 1
NBUF = 2


def _emb_body(seq_len, idx_hbm, table_hbm, out_hbm,
              idx_v, in0, in1, st0, st1, gs0, gs1, ss0, ss1):
    wid = lax.axis_index("s") * NUM_CORES + lax.axis_index("c")

    ins = (in0, in1)
    stages = (st0, st1)
    gsems = (gs0, gs1)
    ssems = (ss0, ss1)

    pltpu.sync_copy(idx_hbm.at[wid], idx_v)

    def gather_desc(s, b):
        return pltpu.make_async_copy(
            table_hbm.at[idx_v.at[s]], ins[b], gsems[b])

    def store_desc(s, b):
        return pltpu.make_async_copy(
            stages[b].at[:, :, pl.ds(0, BBLK)], out_hbm.at[s, :, wid],
            ssems[b])

    for b in range(NBUF):
        gather_desc(b, b).start()

    iota = lax.iota(jnp.int32, LANES)
    cgs = [(iota + 16 * j) >> 3 for j in range(D_MODEL // LANES)]
    c8s = [(iota + 16 * j) & 7 for j in range(D_MODEL // LANES)]

    def outer(s0, _):
        for b in range(NBUF):
            s = s0 * NBUF + b
            inb, stb = ins[b], stages[b]
            gather_desc(s, b).wait()

            @pl.when(s >= NBUF)
            def _():
                store_desc(s - NBUF, b).wait()

            @plsc.parallel_loop(0, BBLK, step=1, unroll=8)
            def row_body(l):
                lane_l = jnp.full((LANES,), l, jnp.int32)
                for j in range(D_MODEL // LANES):
                    v = inb[l, pl.ds(j * LANES, LANES)] * SCALE
                    plsc.store_scatter(stb, [cgs[j], c8s[j], lane_l], v)

            @pl.when(s + NBUF < seq_len)
            def _():
                gather_desc(s + NBUF, b).start()
            store_desc(s, b).start()
        return ()

    lax.fori_loop(0, seq_len // NBUF, outer, ())

    for b in range(NBUF):
        store_desc(seq_len - NBUF + b, b).wait()


@functools.partial(jax.jit, static_argnames=("seq_len",))
def _emb_call(idx, table, seq_len):
    mesh = plsc.VectorSubcoreMesh(core_axis_name="c", subcore_axis_name="s")
    return pl.kernel(
        functools.partial(_emb_body, seq_len),
        mesh=mesh,
        out_type=jax.ShapeDtypeStruct(
            (seq_len, D_MODEL // 8, NUM_WORKERS, 8, BBLK), jnp.float32),
        scratch_types=[
            pltpu.VMEM((seq_len, BBLK), jnp.int32),
            pltpu.VMEM((BBLK, D_MODEL), jnp.float32),
            pltpu.VMEM((BBLK, D_MODEL), jnp.float32),
            pltpu.VMEM((D_MODEL // 8, 8, PITCH), jnp.float32),
            pltpu.VMEM((D_MODEL // 8, 8, PITCH), jnp.float32),
            pltpu.SemaphoreType.DMA,
            pltpu.SemaphoreType.DMA,
            pltpu.SemaphoreType.DMA,
            pltpu.SemaphoreType.DMA,
        ],
        compiler_params=pltpu.CompilerParams(use_tc_tiling_on_sc=False,
                                             needs_layout_passes=False),
    )(idx, table)


def kernel(x, table):
    bsz, seq_len = x.shape
    idx = x.reshape(NUM_WORKERS, BBLK, seq_len).transpose(0, 2, 1)
    out5 = _emb_call(idx, table, seq_len)
    return out5.transpose(2, 4, 0, 1, 3).reshape(bsz, seq_len, D_MODEL)
